# Initial kernel scaffold; baseline (speedup 1.0000x reference)
import jax
import jax.numpy as jnp
from jax.experimental import pallas as pl


def kernel(hours, days, months, years, W_hour, W_day, W_month, W_year):
    raise NotImplementedError("write your pallas kernel here")


# speedup vs baseline: 1.5217x; 1.5217x over previous
import functools

import jax
import jax.numpy as jnp
from jax import lax
from jax.experimental import pallas as pl
from jax.experimental.pallas import tpu as pltpu
from jax.experimental.pallas import tpu_sc as plsc

SEQ = 16384
Q = 32
NC = 2
NS = 16
NW = NC * NS
B_PER_W = SEQ // NW
CHUNK = 128
NCHUNK = B_PER_W // CHUNK
NTAB = 4


def _body(h, d, m, y, wh, wd, wm, wy, out, idx_v, rows_v, sem):
    wid = lax.axis_index("s") * NC + lax.axis_index("c")
    for j, src in enumerate((h, d, m, y)):
        pltpu.sync_copy(src.at[wid], idx_v.at[j])
    copies = []
    for j, tab in enumerate((wh, wd, wm, wy)):
        for k in range(NCHUNK):
            copies.append(
                pltpu.async_copy(
                    tab.at[idx_v.at[j, k]],
                    rows_v.at[j, pl.ds(k * CHUNK, CHUNK)],
                    sem,
                )
            )
    for c in copies:
        c.wait()
    base = wid * B_PER_W
    for j in range(NTAB):
        pltpu.sync_copy(rows_v.at[j], out.at[pl.ds(base, B_PER_W), pl.ds(j * Q, Q)])


_sc_call = pl.kernel(
    _body,
    out_type=jax.ShapeDtypeStruct((SEQ, NTAB * Q), jnp.float32),
    mesh=plsc.VectorSubcoreMesh(core_axis_name="c", subcore_axis_name="s"),
    scratch_types=[
        pltpu.VMEM((NTAB, NCHUNK, CHUNK), jnp.int32),
        pltpu.VMEM((NTAB, B_PER_W, Q), jnp.float32),
        pltpu.SemaphoreType.DMA,
    ],
    compiler_params=pltpu.CompilerParams(use_tc_tiling_on_sc=False),
)


def kernel(hours, days, months, years, W_hour, W_day, W_month, W_year):
    h = hours.astype(jnp.int32).reshape(NW, NCHUNK, CHUNK)
    d = days.astype(jnp.int32).reshape(NW, NCHUNK, CHUNK)
    m = months.astype(jnp.int32).reshape(NW, NCHUNK, CHUNK)
    y = years.astype(jnp.int32).reshape(NW, NCHUNK, CHUNK)
    return _sc_call(h, d, m, y, W_hour, W_day, W_month, W_year)


# speedup vs baseline: 2.1300x; 1.3997x over previous
import jax
import jax.numpy as jnp
from jax import lax
from jax.experimental import pallas as pl
from jax.experimental.pallas import tpu as pltpu
from jax.experimental.pallas import tpu_sc as plsc

SEQ = 16384
Q = 32
D = 4 * Q
NC = 2
NS = 16
NW = NC * NS
B_PER_W = SEQ // NW
CHUNK = 128
NCHUNK = B_PER_W // CHUNK
NTAB = 4


def _body(h, d, m, y, wh, wd, wm, wy, out, idx_v, rows_v, isem, gsem, wsem):
    wid = lax.axis_index("s") * NC + lax.axis_index("c")
    base = wid * B_PER_W
    for c in [pltpu.async_copy(src.at[wid], idx_v.at[j], isem)
              for j, src in enumerate((h, d, m, y))]:
        c.wait()
    tabs = (wh, wd, wm, wy)
    gathers = [
        [
            pltpu.async_copy(
                tabs[j].at[idx_v.at[j, k]],
                rows_v.at[j, pl.ds(k * CHUNK, CHUNK)],
                gsem.at[j],
            )
            for k in range(NCHUNK)
        ]
        for j in range(NTAB)
    ]
    writes = []
    for j in range(NTAB):
        for g in gathers[j]:
            g.wait()
        writes.append(
            pltpu.async_copy(
                rows_v.at[j],
                out.at[pl.ds(base, B_PER_W), pl.ds(j * Q, Q)],
                wsem,
            )
        )
    for c in writes:
        c.wait()


_sc_call = pl.kernel(
    _body,
    out_type=jax.ShapeDtypeStruct((SEQ, D), jnp.float32),
    mesh=plsc.VectorSubcoreMesh(core_axis_name="c", subcore_axis_name="s"),
    scratch_types=[
        pltpu.VMEM((NTAB, NCHUNK, CHUNK), jnp.int32),
        pltpu.VMEM((NTAB, B_PER_W, Q), jnp.float32),
        pltpu.SemaphoreType.DMA,
        pltpu.SemaphoreType.DMA((NTAB,)),
        pltpu.SemaphoreType.DMA,
    ],
    compiler_params=pltpu.CompilerParams(use_tc_tiling_on_sc=False),
)


def kernel(hours, days, months, years, W_hour, W_day, W_month, W_year):
    h = hours.astype(jnp.int32).reshape(NW, NCHUNK, CHUNK)
    d = days.astype(jnp.int32).reshape(NW, NCHUNK, CHUNK)
    m = months.astype(jnp.int32).reshape(NW, NCHUNK, CHUNK)
    y = years.astype(jnp.int32).reshape(NW, NCHUNK, CHUNK)
    return _sc_call(h, d, m, y, W_hour, W_day, W_month, W_year)


# speedup vs baseline: 2.5803x; 1.2114x over previous
import jax
import jax.numpy as jnp
from jax import lax
from jax.experimental import pallas as pl
from jax.experimental.pallas import tpu as pltpu
from jax.experimental.pallas import tpu_sc as plsc

SEQ = 16384
Q = 32
D = 4 * Q
NC = 2
NS = 16
NW = NC * NS
B_PER_W = SEQ // NW
L = 16
NGRP = B_PER_W // L
TAB_ROWS = (24, 7, 12, 10)


def _body(h, d, m, y, wh, wd, wm, wy, out,
          th_v, td_v, tm_v, ty_v, ih_v, id_v, im_v, iy_v, rows_v,
          tsem, isem, wsem):
    wid = lax.axis_index("s") * NC + lax.axis_index("c")
    base = wid * B_PER_W
    tabs_h = (wh, wd, wm, wy)
    tabs_v = (th_v, td_v, tm_v, ty_v)
    idx_h = (h, d, m, y)
    idx_v = (ih_v, id_v, im_v, iy_v)
    copies = [pltpu.async_copy(tabs_h[j], tabs_v[j], tsem) for j in range(4)]
    copies += [pltpu.async_copy(idx_h[j].at[wid], idx_v[j], isem) for j in range(4)]
    for c in copies:
        c.wait()

    lane = lax.iota(jnp.int32, L)
    row_pos = lane * D

    def grp(g, carry):
        pos0 = row_pos + g * (L * D)
        for j in range(4):
            tab_off = idx_v[j][pl.ds(g * L, L)] * Q
            for c in range(Q):
                v = plsc.load_gather(tabs_v[j], [tab_off + c])
                plsc.store_scatter(rows_v, [pos0 + (j * Q + c)], v)
        return carry

    lax.fori_loop(0, NGRP, grp, 0)
    pltpu.async_copy(rows_v, out.at[pl.ds(base * D, B_PER_W * D)], wsem).wait()


_sc_call = pl.kernel(
    _body,
    out_type=jax.ShapeDtypeStruct((SEQ * D,), jnp.float32),
    mesh=plsc.VectorSubcoreMesh(core_axis_name="c", subcore_axis_name="s"),
    scratch_types=[
        pltpu.VMEM((TAB_ROWS[0] * Q,), jnp.float32),
        pltpu.VMEM((TAB_ROWS[1] * Q,), jnp.float32),
        pltpu.VMEM((TAB_ROWS[2] * Q,), jnp.float32),
        pltpu.VMEM((TAB_ROWS[3] * Q,), jnp.float32),
        pltpu.VMEM((B_PER_W,), jnp.int32),
        pltpu.VMEM((B_PER_W,), jnp.int32),
        pltpu.VMEM((B_PER_W,), jnp.int32),
        pltpu.VMEM((B_PER_W,), jnp.int32),
        pltpu.VMEM((B_PER_W * D,), jnp.float32),
        pltpu.SemaphoreType.DMA,
        pltpu.SemaphoreType.DMA,
        pltpu.SemaphoreType.DMA,
    ],
    compiler_params=pltpu.CompilerParams(
        use_tc_tiling_on_sc=False, needs_layout_passes=False),
)


def kernel(hours, days, months, years, W_hour, W_day, W_month, W_year):
    h = hours.astype(jnp.int32).reshape(NW, B_PER_W)
    d = days.astype(jnp.int32).reshape(NW, B_PER_W)
    m = months.astype(jnp.int32).reshape(NW, B_PER_W)
    y = years.astype(jnp.int32).reshape(NW, B_PER_W)
    out = _sc_call(h, d, m, y,
                   W_hour.reshape(-1), W_day.reshape(-1),
                   W_month.reshape(-1), W_year.reshape(-1))
    return out.reshape(SEQ, D)


# speedup vs baseline: 3.4695x; 1.3446x over previous
import jax
import jax.numpy as jnp
from jax import lax
from jax.experimental import pallas as pl
from jax.experimental.pallas import tpu as pltpu
from jax.experimental.pallas import tpu_sc as plsc

SEQ = 16384
Q = 32
D = 4 * Q
NC = 2
NS = 16
NW = NC * NS
B_PER_W = SEQ // NW
L = 16
NGRP = B_PER_W // L
TAB_ROWS = (24, 7, 12, 10)


def _body(h, d, m, y, wh, wd, wm, wy, out,
          th_v, td_v, tm_v, ty_v, ih_v, id_v, im_v, iy_v, rows_v,
          tsem, isem, wsem):
    wid = lax.axis_index("s") * NC + lax.axis_index("c")
    base = wid * B_PER_W
    tabs_h = (wh, wd, wm, wy)
    tabs_v = (th_v, td_v, tm_v, ty_v)
    idx_h = (h, d, m, y)
    idx_v = (ih_v, id_v, im_v, iy_v)
    copies = [pltpu.async_copy(tabs_h[j], tabs_v[j], tsem) for j in range(4)]
    copies += [pltpu.async_copy(idx_h[j].at[wid], idx_v[j], isem) for j in range(4)]
    for c in copies:
        c.wait()

    lane = lax.iota(jnp.int32, L)
    row_pos = lane * D

    @plsc.parallel_loop(0, NGRP)
    def grp(g):
        pos0 = row_pos + g * (L * D)
        for j in range(4):
            tab_off = idx_v[j][pl.ds(g * L, L)] * Q
            for c in range(Q):
                v = plsc.load_gather(tabs_v[j], [tab_off + c])
                plsc.store_scatter(rows_v, [pos0 + (j * Q + c)], v)
    pltpu.async_copy(rows_v, out.at[pl.ds(base * D, B_PER_W * D)], wsem).wait()


_sc_call = pl.kernel(
    _body,
    out_type=jax.ShapeDtypeStruct((SEQ * D,), jnp.float32),
    mesh=plsc.VectorSubcoreMesh(core_axis_name="c", subcore_axis_name="s"),
    scratch_types=[
        pltpu.VMEM((TAB_ROWS[0] * Q,), jnp.float32),
        pltpu.VMEM((TAB_ROWS[1] * Q,), jnp.float32),
        pltpu.VMEM((TAB_ROWS[2] * Q,), jnp.float32),
        pltpu.VMEM((TAB_ROWS[3] * Q,), jnp.float32),
        pltpu.VMEM((B_PER_W,), jnp.int32),
        pltpu.VMEM((B_PER_W,), jnp.int32),
        pltpu.VMEM((B_PER_W,), jnp.int32),
        pltpu.VMEM((B_PER_W,), jnp.int32),
        pltpu.VMEM((B_PER_W * D,), jnp.float32),
        pltpu.SemaphoreType.DMA,
        pltpu.SemaphoreType.DMA,
        pltpu.SemaphoreType.DMA,
    ],
    compiler_params=pltpu.CompilerParams(
        use_tc_tiling_on_sc=False, needs_layout_passes=False),
)


def kernel(hours, days, months, years, W_hour, W_day, W_month, W_year):
    h = hours.astype(jnp.int32).reshape(NW, B_PER_W)
    d = days.astype(jnp.int32).reshape(NW, B_PER_W)
    m = months.astype(jnp.int32).reshape(NW, B_PER_W)
    y = years.astype(jnp.int32).reshape(NW, B_PER_W)
    out = _sc_call(h, d, m, y,
                   W_hour.reshape(-1), W_day.reshape(-1),
                   W_month.reshape(-1), W_year.reshape(-1))
    return out.reshape(SEQ, D)
